# Initial kernel scaffold; baseline (speedup 1.0000x reference)
import jax
import jax.numpy as jnp
from jax.experimental import pallas as pl


def kernel(x, src, dst, edge_vec, W1, W2):
    raise NotImplementedError("write your pallas kernel here")


# speedup vs baseline: 4.5245x; 4.5245x over previous
import functools
import numpy as np
import jax
import jax.numpy as jnp
from jax.experimental import pallas as pl
from jax.experimental.pallas import tpu as pltpu

_M0 = 32
_M1 = 32
_NUM_BASIS = 10
_MAX_RADIUS = 4.5
_BLOCK_E = 256


def _dense_body(evT_ref, xsT_ref, w1_ref, w2t_ref, outT_ref):
    evT = evT_ref[...]
    d2 = jnp.sum(evT * evT, axis=0, keepdims=True)
    d = jnp.sqrt(d2)
    step = _MAX_RADIUS / (_NUM_BASIS + 1)
    B = evT.shape[1]
    basis = (jax.lax.broadcasted_iota(jnp.int32, (_NUM_BASIS, B), 0) + 1
             ).astype(jnp.float32) * step
    diff = (d - basis) * (1.0 / step)

    def sus(t):
        ts = jnp.where(t > 0.0, t, 1.0)
        return jnp.where(t > 0.0, jnp.exp(-1.0 / ts), 0.0)

    g = sus(diff + 1.0) * sus(1.0 - diff)
    h = jax.nn.relu(
        jax.lax.dot_general(w1_ref[...], g, (((0,), (0,)), ((), ())),
                            preferred_element_type=jnp.float32))
    w = jax.lax.dot_general(w2t_ref[...], h, (((1,), (0,)), ((), ())),
                            preferred_element_type=jnp.float32)

    xsT = xsT_ref[...]
    x0 = xsT[:_M0]
    x1k = [xsT[_M0 + 32 * k:_M0 + 32 * (k + 1)] for k in range(3)]
    dinv = 1.0 / jnp.maximum(d, 1e-12)
    n = [evT[k:k + 1] * dinv for k in range(3)]
    dot = x1k[0] * n[0] + x1k[1] * n[1] + x1k[2] * n[2]

    def contract(wblk, vec):
        p = wblk.reshape(_M0, _M0, B) * vec[:, None, :]
        return jnp.sum(p, axis=0)

    t1 = contract(w[0:1024], x0)
    t2 = contract(w[1024:2048], x0)
    t4 = contract(w[3072:4096], dot)
    out0 = t1 + t4
    w3 = w[2048:3072].reshape(_M0, _M0, B)
    outs = [out0]
    for k in range(3):
        t3k = jnp.sum(w3 * x1k[k][:, None, :], axis=0)
        outs.append(t2 * n[k] + t3k)
    outT_ref[...] = jnp.concatenate(outs, axis=0)


def _dense_edges(evT, xsT, w1s, w2ts, e_pad):
    nblk = e_pad // _BLOCK_E
    return pl.pallas_call(
        _dense_body,
        grid=(nblk,),
        in_specs=[
            pl.BlockSpec((3, _BLOCK_E), lambda i: (0, i)),
            pl.BlockSpec((128, _BLOCK_E), lambda i: (0, i)),
            pl.BlockSpec((_NUM_BASIS, 64), lambda i: (0, 0)),
            pl.BlockSpec((4096, 64), lambda i: (0, 0)),
        ],
        out_specs=pl.BlockSpec((128, _BLOCK_E), lambda i: (0, i)),
        out_shape=jax.ShapeDtypeStruct((128, e_pad), jnp.float32),
    )(evT, xsT, w1s, w2ts)


@jax.jit
def kernel(x, src, dst, edge_vec, W1, W2):
    N = x.shape[0]
    E = src.shape[0]
    e_pad = ((E + _BLOCK_E - 1) // _BLOCK_E) * _BLOCK_E

    c_out = (1.0 / (_M0 + _M1)) ** 0.5
    w1s = W1 * np.float32(1.14136 * np.exp(2.0))
    scale = np.float32((2.0 ** 0.5) / (64.0 ** 0.5) * c_out / (E / N))
    w2s = W2 * scale
    o1, o2 = _M0 * _M0, _M0 * _M0 + _M0 * _M1
    w2s = w2s.at[:, o1:o2].mul(np.float32(3.0 ** 0.5))
    w2ts = w2s.T

    u = np.arange(_M1)
    perm_in = np.concatenate(
        [np.arange(_M0)] + [_M0 + 3 * u + k for k in range(3)]).astype(np.int32)
    x_perm = x[:, perm_in]

    src_pad = jnp.pad(src, (0, e_pad - E))
    xsT = x_perm[src_pad].T
    evT = jnp.pad(edge_vec[:, np.array([1, 2, 0])],
                  ((0, e_pad - E), (0, 0)), constant_values=1.0).T

    outT = _dense_edges(evT, xsT, w1s, w2ts, e_pad)

    summand = outT.T[:E]
    out_km = jax.ops.segment_sum(summand, dst, num_segments=N)

    perm_out = np.concatenate(
        [np.arange(_M0)] + [_M0 + 3 * u + k for k in range(3)]).astype(np.int32)
    inv = np.empty_like(perm_out)
    inv[perm_out] = np.arange(128, dtype=np.int32)
    return out_km[:, inv]


# speedup vs baseline: 4.7934x; 1.0594x over previous
import functools
import numpy as np
import jax
import jax.numpy as jnp
from jax.experimental import pallas as pl
from jax.experimental.pallas import tpu as pltpu
from jax.experimental.pallas import tpu_sc as plsc

_M0 = 32
_M1 = 32
_NUM_BASIS = 10
_MAX_RADIUS = 4.5
_BLOCK_E = 256

_NC = 2
_NS = 16
_NW = _NC * _NS
_SCAT_CHUNK = 128
_N_PAD = 10112


def _dense_body(evT_ref, xsT_ref, w1_ref, w2t_ref, outT_ref):
    evT = evT_ref[...]
    d2 = jnp.sum(evT * evT, axis=0, keepdims=True)
    d = jnp.sqrt(d2)
    step = _MAX_RADIUS / (_NUM_BASIS + 1)
    B = evT.shape[1]
    basis = (jax.lax.broadcasted_iota(jnp.int32, (_NUM_BASIS, B), 0) + 1
             ).astype(jnp.float32) * step
    diff = (d - basis) * (1.0 / step)

    def sus(t):
        ts = jnp.where(t > 0.0, t, 1.0)
        return jnp.where(t > 0.0, jnp.exp(-1.0 / ts), 0.0)

    g = sus(diff + 1.0) * sus(1.0 - diff)
    h = jax.nn.relu(
        jax.lax.dot_general(w1_ref[...], g, (((0,), (0,)), ((), ())),
                            preferred_element_type=jnp.float32))
    w = jax.lax.dot_general(w2t_ref[...], h, (((1,), (0,)), ((), ())),
                            preferred_element_type=jnp.float32)

    xsT = xsT_ref[...]
    x0 = xsT[:_M0]
    x1k = [xsT[_M0 + 32 * k:_M0 + 32 * (k + 1)] for k in range(3)]
    dinv = 1.0 / jnp.maximum(d, 1e-12)
    n = [evT[k:k + 1] * dinv for k in range(3)]
    dot = x1k[0] * n[0] + x1k[1] * n[1] + x1k[2] * n[2]

    def contract(wblk, vec):
        p = wblk.reshape(_M0, _M0, B) * vec[:, None, :]
        return jnp.sum(p, axis=0)

    t1 = contract(w[0:1024], x0)
    t2 = contract(w[1024:2048], x0)
    t4 = contract(w[3072:4096], dot)
    out0 = t1 + t4
    w3 = w[2048:3072].reshape(_M0, _M0, B)
    outs = [out0]
    for k in range(3):
        t3k = jnp.sum(w3 * x1k[k][:, None, :], axis=0)
        outs.append(t2 * n[k] + t3k)
    outT_ref[...] = jnp.concatenate(outs, axis=0)


def _dense_edges(evT, xsT, w1s, w2ts, e_pad):
    nblk = e_pad // _BLOCK_E
    return pl.pallas_call(
        _dense_body,
        grid=(nblk,),
        in_specs=[
            pl.BlockSpec((3, _BLOCK_E), lambda i: (0, i)),
            pl.BlockSpec((128, _BLOCK_E), lambda i: (0, i)),
            pl.BlockSpec((_NUM_BASIS, 64), lambda i: (0, 0)),
            pl.BlockSpec((4096, 64), lambda i: (0, 0)),
        ],
        out_specs=pl.BlockSpec((128, _BLOCK_E), lambda i: (0, i)),
        out_shape=jax.ShapeDtypeStruct((128, e_pad), jnp.float32),
    )(evT, xsT, w1s, w2ts)


def _scatter_add(summand, dst3d, zeros, e_pad):
    per_w = e_pad // _NW
    nchunk = per_w // _SCAT_CHUNK
    rpt = _N_PAD // _NS
    mesh = plsc.VectorSubcoreMesh(core_axis_name="c", subcore_axis_name="s")

    @functools.partial(
        pl.kernel, mesh=mesh,
        out_type=jax.ShapeDtypeStruct((_NC, _N_PAD, 128), jnp.float32),
        scratch_types=[
            pltpu.VMEM((nchunk, _SCAT_CHUNK), jnp.int32),
            pltpu.VMEM((_SCAT_CHUNK, 128), jnp.float32),
            pltpu.VMEM_SHARED((_N_PAD, 128), jnp.float32),
        ],
    )
    def scat(summand_hbm, dst_hbm, zeros_hbm, out_hbm, idx_v, data_v, acc):
        c = jax.lax.axis_index("c")
        s = jax.lax.axis_index("s")
        pltpu.sync_copy(zeros_hbm, acc.at[pl.ds(s * rpt, rpt)])
        plsc.subcore_barrier()
        wid = c * _NS + s
        pltpu.sync_copy(dst_hbm.at[wid], idx_v)
        base = wid * per_w
        for j in range(nchunk):
            pltpu.sync_copy(
                summand_hbm.at[pl.ds(base + j * _SCAT_CHUNK, _SCAT_CHUNK)], data_v)
            pltpu.sync_copy(data_v, acc.at[idx_v.at[j]], add=True)
        plsc.subcore_barrier()
        pltpu.sync_copy(acc.at[pl.ds(s * rpt, rpt)],
                        out_hbm.at[c, pl.ds(s * rpt, rpt)])

    return scat(summand, dst3d, zeros)


def _combine_body(a_ref, b_ref, o_ref):
    o_ref[...] = a_ref[...] + b_ref[...]


def _combine(parts):
    return pl.pallas_call(
        _combine_body,
        out_shape=jax.ShapeDtypeStruct((_N_PAD, 128), jnp.float32),
    )(parts[0], parts[1])


@jax.jit
def kernel(x, src, dst, edge_vec, W1, W2):
    N = x.shape[0]
    E = src.shape[0]
    grain = _NW * _SCAT_CHUNK
    e_pad = ((E + grain - 1) // grain) * grain

    c_out = (1.0 / (_M0 + _M1)) ** 0.5
    w1s = W1 * np.float32(1.14136 * np.exp(2.0))
    scale = np.float32((2.0 ** 0.5) / (64.0 ** 0.5) * c_out / (E / N))
    w2s = W2 * scale
    o1, o2 = _M0 * _M0, _M0 * _M0 + _M0 * _M1
    w2s = w2s.at[:, o1:o2].mul(np.float32(3.0 ** 0.5))
    w2ts = w2s.T

    u = np.arange(_M1)
    perm_in = np.concatenate(
        [np.arange(_M0)] + [_M0 + 3 * u + k for k in range(3)]).astype(np.int32)
    x_perm = x[:, perm_in]

    src_pad = jnp.pad(src, (0, e_pad - E))
    xsT = x_perm[src_pad].T
    evT = jnp.pad(edge_vec[:, np.array([1, 2, 0])],
                  ((0, e_pad - E), (0, 0)), constant_values=1.0).T

    outT = _dense_edges(evT, xsT, w1s, w2ts, e_pad)

    summand = outT.T
    dst3d = jnp.pad(dst, (0, e_pad - E), constant_values=_N_PAD - 8
                    ).reshape(_NW, e_pad // (_NW * _SCAT_CHUNK), _SCAT_CHUNK)
    zeros = jnp.zeros((_N_PAD // _NS, 128), jnp.float32)
    parts = _scatter_add(summand, dst3d, zeros, e_pad)
    out_km = _combine(parts)[:N]

    perm_out = np.concatenate(
        [np.arange(_M0)] + [_M0 + 3 * u + k for k in range(3)]).astype(np.int32)
    inv = np.empty_like(perm_out)
    inv[perm_out] = np.arange(128, dtype=np.int32)
    return out_km[:, inv]


# speedup vs baseline: 5.5389x; 1.1555x over previous
import functools
import numpy as np
import jax
import jax.numpy as jnp
from jax.experimental import pallas as pl
from jax.experimental.pallas import tpu as pltpu
from jax.experimental.pallas import tpu_sc as plsc

_M0 = 32
_M1 = 32
_NUM_BASIS = 10
_MAX_RADIUS = 4.5
_BLOCK_E = 256

_NC = 2
_NS = 16
_NW = _NC * _NS
_SCAT_CHUNK = 128
_N_PAD = 10112


def _dense_body(evT_ref, xsT_ref, w1_ref, w2t_ref, outT_ref):
    evT = evT_ref[...]
    d2 = jnp.sum(evT * evT, axis=0, keepdims=True)
    d = jnp.sqrt(d2)
    step = _MAX_RADIUS / (_NUM_BASIS + 1)
    B = evT.shape[1]
    basis = (jax.lax.broadcasted_iota(jnp.int32, (_NUM_BASIS, B), 0) + 1
             ).astype(jnp.float32) * step
    diff = (d - basis) * (1.0 / step)

    def sus(t):
        ts = jnp.where(t > 0.0, t, 1.0)
        return jnp.where(t > 0.0, jnp.exp(-1.0 / ts), 0.0)

    g = sus(diff + 1.0) * sus(1.0 - diff)
    h = jax.nn.relu(
        jax.lax.dot_general(w1_ref[...], g, (((0,), (0,)), ((), ())),
                            preferred_element_type=jnp.float32))
    w = jax.lax.dot_general(w2t_ref[...], h, (((1,), (0,)), ((), ())),
                            preferred_element_type=jnp.float32)

    xsT = xsT_ref[...]
    x0 = xsT[:_M0]
    x1k = [xsT[_M0 + 32 * k:_M0 + 32 * (k + 1)] for k in range(3)]
    dinv = 1.0 / jnp.maximum(d, 1e-12)
    n = [evT[k:k + 1] * dinv for k in range(3)]
    dot = x1k[0] * n[0] + x1k[1] * n[1] + x1k[2] * n[2]

    def contract(wblk, vec):
        p = wblk.reshape(_M0, _M0, B) * vec[:, None, :]
        return jnp.sum(p, axis=0)

    t1 = contract(w[0:1024], x0)
    t2 = contract(w[1024:2048], x0)
    t4 = contract(w[3072:4096], dot)
    out0 = t1 + t4
    w3 = w[2048:3072].reshape(_M0, _M0, B)
    outs = [out0]
    for k in range(3):
        t3k = jnp.sum(w3 * x1k[k][:, None, :], axis=0)
        outs.append(t2 * n[k] + t3k)
    outT_ref[...] = jnp.concatenate(outs, axis=0)


def _dense_edges(evT, xsT, w1s, w2ts, e_pad):
    nblk = e_pad // _BLOCK_E
    return pl.pallas_call(
        _dense_body,
        grid=(nblk,),
        in_specs=[
            pl.BlockSpec((3, _BLOCK_E), lambda i: (0, i)),
            pl.BlockSpec((128, _BLOCK_E), lambda i: (0, i)),
            pl.BlockSpec((_NUM_BASIS, 64), lambda i: (0, 0)),
            pl.BlockSpec((4096, 64), lambda i: (0, 0)),
        ],
        out_specs=pl.BlockSpec((128, _BLOCK_E), lambda i: (0, i)),
        out_shape=jax.ShapeDtypeStruct((128, e_pad), jnp.float32),
    )(evT, xsT, w1s, w2ts)


def _gather_rows(table, src3d, e_pad):
    per_w = e_pad // _NW
    nchunk = per_w // _SCAT_CHUNK
    mesh = plsc.VectorSubcoreMesh(core_axis_name="c", subcore_axis_name="s")

    @functools.partial(
        pl.kernel, mesh=mesh,
        out_type=jax.ShapeDtypeStruct((e_pad, 128), jnp.float32),
        scratch_types=[
            pltpu.VMEM((nchunk, _SCAT_CHUNK), jnp.int32),
            pltpu.VMEM((_SCAT_CHUNK, 128), jnp.float32),
            pltpu.VMEM((_SCAT_CHUNK, 128), jnp.float32),
            pltpu.SemaphoreType.DMA,
            pltpu.SemaphoreType.DMA,
        ],
    )
    def gat(table_hbm, src_hbm, out_hbm, idx_v, buf0, buf1, sem0, sem1):
        c = jax.lax.axis_index("c")
        s = jax.lax.axis_index("s")
        wid = c * _NS + s
        pltpu.sync_copy(src_hbm.at[wid], idx_v)
        base = wid * per_w
        bufs = (buf0, buf1)
        sems = (sem0, sem1)
        copies = [None] * nchunk
        copies[0] = pltpu.async_copy(table_hbm.at[idx_v.at[0]], buf0, sem0)
        for j in range(nchunk):
            copies[j].wait()
            if j + 1 < nchunk:
                copies[j + 1] = pltpu.async_copy(
                    table_hbm.at[idx_v.at[j + 1]], bufs[(j + 1) % 2],
                    sems[(j + 1) % 2])
            pltpu.sync_copy(bufs[j % 2],
                            out_hbm.at[pl.ds(base + j * _SCAT_CHUNK, _SCAT_CHUNK)])

    return gat(table, src3d)


def _scatter_add(summand, dst3d, zeros, e_pad):
    per_w = e_pad // _NW
    nchunk = per_w // _SCAT_CHUNK
    rpt = _N_PAD // _NS
    mesh = plsc.VectorSubcoreMesh(core_axis_name="c", subcore_axis_name="s")

    @functools.partial(
        pl.kernel, mesh=mesh,
        out_type=jax.ShapeDtypeStruct((_NC, _N_PAD, 128), jnp.float32),
        scratch_types=[
            pltpu.VMEM((nchunk, _SCAT_CHUNK), jnp.int32),
            pltpu.VMEM((_SCAT_CHUNK, 128), jnp.float32),
            pltpu.VMEM_SHARED((_N_PAD, 128), jnp.float32),
        ],
    )
    def scat(summand_hbm, dst_hbm, zeros_hbm, out_hbm, idx_v, data_v, acc):
        c = jax.lax.axis_index("c")
        s = jax.lax.axis_index("s")
        pltpu.sync_copy(zeros_hbm, acc.at[pl.ds(s * rpt, rpt)])
        plsc.subcore_barrier()
        wid = c * _NS + s
        pltpu.sync_copy(dst_hbm.at[wid], idx_v)
        base = wid * per_w
        for j in range(nchunk):
            pltpu.sync_copy(
                summand_hbm.at[pl.ds(base + j * _SCAT_CHUNK, _SCAT_CHUNK)], data_v)
            pltpu.sync_copy(data_v, acc.at[idx_v.at[j]], add=True)
        plsc.subcore_barrier()
        pltpu.sync_copy(acc.at[pl.ds(s * rpt, rpt)],
                        out_hbm.at[c, pl.ds(s * rpt, rpt)])

    return scat(summand, dst3d, zeros)


def _combine_body(a_ref, b_ref, o_ref):
    o_ref[...] = a_ref[...] + b_ref[...]


def _combine(parts):
    return pl.pallas_call(
        _combine_body,
        out_shape=jax.ShapeDtypeStruct((_N_PAD, 128), jnp.float32),
    )(parts[0], parts[1])


@jax.jit
def kernel(x, src, dst, edge_vec, W1, W2):
    N = x.shape[0]
    E = src.shape[0]
    grain = _NW * _SCAT_CHUNK
    e_pad = ((E + grain - 1) // grain) * grain

    c_out = (1.0 / (_M0 + _M1)) ** 0.5
    w1s = W1 * np.float32(1.14136 * np.exp(2.0))
    scale = np.float32((2.0 ** 0.5) / (64.0 ** 0.5) * c_out / (E / N))
    w2s = W2 * scale
    o1, o2 = _M0 * _M0, _M0 * _M0 + _M0 * _M1
    w2s = w2s.at[:, o1:o2].mul(np.float32(3.0 ** 0.5))
    w2ts = w2s.T

    u = np.arange(_M1)
    perm_in = np.concatenate(
        [np.arange(_M0)] + [_M0 + 3 * u + k for k in range(3)]).astype(np.int32)
    x_perm = x[:, perm_in]

    src3d = jnp.pad(src, (0, e_pad - E)).reshape(
        _NW, e_pad // (_NW * _SCAT_CHUNK), _SCAT_CHUNK)
    xs = _gather_rows(x_perm, src3d, e_pad)
    xsT = xs.T
    evT = jnp.pad(edge_vec[:, np.array([1, 2, 0])],
                  ((0, e_pad - E), (0, 0)), constant_values=1.0).T

    outT = _dense_edges(evT, xsT, w1s, w2ts, e_pad)

    summand = outT.T
    dst3d = jnp.pad(dst, (0, e_pad - E), constant_values=_N_PAD - 8
                    ).reshape(_NW, e_pad // (_NW * _SCAT_CHUNK), _SCAT_CHUNK)
    zeros = jnp.zeros((_N_PAD // _NS, 128), jnp.float32)
    parts = _scatter_add(summand, dst3d, zeros, e_pad)
    out_km = _combine(parts)[:N]

    perm_out = np.concatenate(
        [np.arange(_M0)] + [_M0 + 3 * u + k for k in range(3)]).astype(np.int32)
    inv = np.empty_like(perm_out)
    inv[perm_out] = np.arange(128, dtype=np.int32)
    return out_km[:, inv]


# speedup vs baseline: 5.6752x; 1.0246x over previous
import functools
import numpy as np
import jax
import jax.numpy as jnp
from jax.experimental import pallas as pl
from jax.experimental.pallas import tpu as pltpu
from jax.experimental.pallas import tpu_sc as plsc

_M0 = 32
_M1 = 32
_NUM_BASIS = 10
_MAX_RADIUS = 4.5
_BLOCK_E = 256

_NC = 2
_NS = 16
_NW = _NC * _NS
_SCAT_CHUNK = 128
_N_PAD = 10112


def _dense_body(evT_ref, xsT_ref, w1_ref, w2t_ref, outT_ref):
    evT = evT_ref[...]
    d2 = jnp.sum(evT * evT, axis=0, keepdims=True)
    d = jnp.sqrt(d2)
    step = _MAX_RADIUS / (_NUM_BASIS + 1)
    B = evT.shape[1]
    basis = (jax.lax.broadcasted_iota(jnp.int32, (_NUM_BASIS, B), 0) + 1
             ).astype(jnp.float32) * step
    diff = (d - basis) * (1.0 / step)

    def sus(t):
        ts = jnp.where(t > 0.0, t, 1.0)
        return jnp.where(t > 0.0, jnp.exp(-1.0 / ts), 0.0)

    g = sus(diff + 1.0) * sus(1.0 - diff)
    h = jax.nn.relu(
        jax.lax.dot_general(w1_ref[...], g, (((0,), (0,)), ((), ())),
                            preferred_element_type=jnp.float32))
    w = jax.lax.dot_general(w2t_ref[...], h.astype(jnp.bfloat16),
                            (((1,), (0,)), ((), ())),
                            preferred_element_type=jnp.float32)

    xsT = xsT_ref[...]
    x0 = xsT[:_M0]
    x1k = [xsT[_M0 + 32 * k:_M0 + 32 * (k + 1)] for k in range(3)]
    dinv = 1.0 / jnp.maximum(d, 1e-12)
    n = [evT[k:k + 1] * dinv for k in range(3)]
    dot = x1k[0] * n[0] + x1k[1] * n[1] + x1k[2] * n[2]

    def contract(wblk, vec):
        p = wblk.reshape(_M0, _M0, B) * vec[:, None, :]
        return jnp.sum(p, axis=0)

    t1 = contract(w[0:1024], x0)
    t2 = contract(w[1024:2048], x0)
    t4 = contract(w[3072:4096], dot)
    out0 = t1 + t4
    w3 = w[2048:3072].reshape(_M0, _M0, B)
    outs = [out0]
    for k in range(3):
        t3k = jnp.sum(w3 * x1k[k][:, None, :], axis=0)
        outs.append(t2 * n[k] + t3k)
    outT_ref[...] = jnp.concatenate(outs, axis=0)


def _dense_edges(evT, xsT, w1s, w2ts, e_pad):
    nblk = e_pad // _BLOCK_E
    return pl.pallas_call(
        _dense_body,
        grid=(nblk,),
        in_specs=[
            pl.BlockSpec((3, _BLOCK_E), lambda i: (0, i)),
            pl.BlockSpec((128, _BLOCK_E), lambda i: (0, i)),
            pl.BlockSpec((_NUM_BASIS, 64), lambda i: (0, 0)),
            pl.BlockSpec((4096, 64), lambda i: (0, 0)),
        ],
        out_specs=pl.BlockSpec((128, _BLOCK_E), lambda i: (0, i)),
        out_shape=jax.ShapeDtypeStruct((128, e_pad), jnp.float32),
    )(evT, xsT, w1s, w2ts)


def _gather_rows(table, src3d, e_pad):
    per_w = e_pad // _NW
    nchunk = per_w // _SCAT_CHUNK
    mesh = plsc.VectorSubcoreMesh(core_axis_name="c", subcore_axis_name="s")

    @functools.partial(
        pl.kernel, mesh=mesh,
        out_type=jax.ShapeDtypeStruct((e_pad, 128), jnp.float32),
        scratch_types=[
            pltpu.VMEM((nchunk, _SCAT_CHUNK), jnp.int32),
            pltpu.VMEM((_SCAT_CHUNK, 128), jnp.float32),
            pltpu.VMEM((_SCAT_CHUNK, 128), jnp.float32),
            pltpu.SemaphoreType.DMA,
            pltpu.SemaphoreType.DMA,
        ],
    )
    def gat(table_hbm, src_hbm, out_hbm, idx_v, buf0, buf1, sem0, sem1):
        c = jax.lax.axis_index("c")
        s = jax.lax.axis_index("s")
        wid = c * _NS + s
        pltpu.sync_copy(src_hbm.at[wid], idx_v)
        base = wid * per_w
        bufs = (buf0, buf1)
        sems = (sem0, sem1)
        copies = [None] * nchunk
        copies[0] = pltpu.async_copy(table_hbm.at[idx_v.at[0]], buf0, sem0)
        for j in range(nchunk):
            copies[j].wait()
            if j + 1 < nchunk:
                copies[j + 1] = pltpu.async_copy(
                    table_hbm.at[idx_v.at[j + 1]], bufs[(j + 1) % 2],
                    sems[(j + 1) % 2])
            pltpu.sync_copy(bufs[j % 2],
                            out_hbm.at[pl.ds(base + j * _SCAT_CHUNK, _SCAT_CHUNK)])

    return gat(table, src3d)


def _scatter_add(summand, dst3d, zeros, e_pad):
    per_w = e_pad // _NW
    nchunk = per_w // _SCAT_CHUNK
    rpt = _N_PAD // _NS
    mesh = plsc.VectorSubcoreMesh(core_axis_name="c", subcore_axis_name="s")

    @functools.partial(
        pl.kernel, mesh=mesh,
        out_type=jax.ShapeDtypeStruct((_NC, _N_PAD, 128), jnp.float32),
        scratch_types=[
            pltpu.VMEM((nchunk, _SCAT_CHUNK), jnp.int32),
            pltpu.VMEM((_SCAT_CHUNK, 128), jnp.float32),
            pltpu.VMEM_SHARED((_N_PAD, 128), jnp.float32),
        ],
    )
    def scat(summand_hbm, dst_hbm, zeros_hbm, out_hbm, idx_v, data_v, acc):
        c = jax.lax.axis_index("c")
        s = jax.lax.axis_index("s")
        pltpu.sync_copy(zeros_hbm, acc.at[pl.ds(s * rpt, rpt)])
        plsc.subcore_barrier()
        wid = c * _NS + s
        pltpu.sync_copy(dst_hbm.at[wid], idx_v)
        base = wid * per_w
        for j in range(nchunk):
            pltpu.sync_copy(
                summand_hbm.at[pl.ds(base + j * _SCAT_CHUNK, _SCAT_CHUNK)], data_v)
            pltpu.sync_copy(data_v, acc.at[idx_v.at[j]], add=True)
        plsc.subcore_barrier()
        pltpu.sync_copy(acc.at[pl.ds(s * rpt, rpt)],
                        out_hbm.at[c, pl.ds(s * rpt, rpt)])

    return scat(summand, dst3d, zeros)


def _combine_body(a_ref, b_ref, o_ref):
    o_ref[...] = a_ref[...] + b_ref[...]


def _combine(parts):
    return pl.pallas_call(
        _combine_body,
        out_shape=jax.ShapeDtypeStruct((_N_PAD, 128), jnp.float32),
    )(parts[0], parts[1])


@jax.jit
def kernel(x, src, dst, edge_vec, W1, W2):
    N = x.shape[0]
    E = src.shape[0]
    grain = _NW * _SCAT_CHUNK
    e_pad = ((E + grain - 1) // grain) * grain

    c_out = (1.0 / (_M0 + _M1)) ** 0.5
    w1s = W1 * np.float32(1.14136 * np.exp(2.0))
    scale = np.float32((2.0 ** 0.5) / (64.0 ** 0.5) * c_out / (E / N))
    w2s = W2 * scale
    o1, o2 = _M0 * _M0, _M0 * _M0 + _M0 * _M1
    w2s = w2s.at[:, o1:o2].mul(np.float32(3.0 ** 0.5))
    w2ts = w2s.T.astype(jnp.bfloat16)

    u = np.arange(_M1)
    perm_in = np.concatenate(
        [np.arange(_M0)] + [_M0 + 3 * u + k for k in range(3)]).astype(np.int32)
    x_perm = x[:, perm_in]

    src3d = jnp.pad(src, (0, e_pad - E)).reshape(
        _NW, e_pad // (_NW * _SCAT_CHUNK), _SCAT_CHUNK)
    xs = _gather_rows(x_perm, src3d, e_pad)
    xsT = xs.T
    evT = jnp.pad(edge_vec[:, np.array([1, 2, 0])],
                  ((0, e_pad - E), (0, 0)), constant_values=1.0).T

    outT = _dense_edges(evT, xsT, w1s, w2ts, e_pad)

    summand = outT.T
    dst3d = jnp.pad(dst, (0, e_pad - E), constant_values=_N_PAD - 8
                    ).reshape(_NW, e_pad // (_NW * _SCAT_CHUNK), _SCAT_CHUNK)
    zeros = jnp.zeros((_N_PAD // _NS, 128), jnp.float32)
    parts = _scatter_add(summand, dst3d, zeros, e_pad)
    out_km = _combine(parts)[:N]

    perm_out = np.concatenate(
        [np.arange(_M0)] + [_M0 + 3 * u + k for k in range(3)]).astype(np.int32)
    inv = np.empty_like(perm_out)
    inv[perm_out] = np.arange(128, dtype=np.int32)
    return out_km[:, inv]


# speedup vs baseline: 5.7971x; 1.0215x over previous
import functools
import numpy as np
import jax
import jax.numpy as jnp
from jax.experimental import pallas as pl
from jax.experimental.pallas import tpu as pltpu
from jax.experimental.pallas import tpu_sc as plsc

_M0 = 32
_M1 = 32
_NUM_BASIS = 10
_MAX_RADIUS = 4.5
_BLOCK_E = 256

_NC = 2
_NS = 16
_NW = _NC * _NS
_SCAT_CHUNK = 128
_N_PAD = 10112


def _dense_body(evT_ref, xsT_ref, w1_ref, w2t_ref, outT_ref):
    evT = evT_ref[...]
    d2 = jnp.sum(evT * evT, axis=0, keepdims=True)
    d = jnp.sqrt(d2)
    step = _MAX_RADIUS / (_NUM_BASIS + 1)
    B = evT.shape[1]
    basis = (jax.lax.broadcasted_iota(jnp.int32, (_NUM_BASIS, B), 0) + 1
             ).astype(jnp.float32) * step
    diff = (d - basis) * (1.0 / step)

    def sus(t):
        ts = jnp.where(t > 0.0, t, 1.0)
        return jnp.where(t > 0.0, jnp.exp(-1.0 / ts), 0.0)

    g = sus(diff + 1.0) * sus(1.0 - diff)
    h = jax.nn.relu(
        jax.lax.dot_general(w1_ref[...], g, (((0,), (0,)), ((), ())),
                            preferred_element_type=jnp.float32))
    w = jax.lax.dot_general(w2t_ref[...], h.astype(jnp.bfloat16),
                            (((1,), (0,)), ((), ())),
                            preferred_element_type=jnp.float32)

    xsT = xsT_ref[...]
    x0 = xsT[:_M0]
    x1k = [xsT[_M0 + 32 * k:_M0 + 32 * (k + 1)] for k in range(3)]
    dinv = 1.0 / jnp.maximum(d, 1e-12)
    n = [evT[k:k + 1] * dinv for k in range(3)]
    dot = x1k[0] * n[0] + x1k[1] * n[1] + x1k[2] * n[2]

    def contract(wblk, vec):
        p = wblk.reshape(_M0, _M0, B) * vec[:, None, :]
        return jnp.sum(p, axis=0)

    t1 = contract(w[0:1024], x0)
    t2 = contract(w[1024:2048], x0)
    t4 = contract(w[3072:4096], dot)
    out0 = t1 + t4
    w3 = w[2048:3072].reshape(_M0, _M0, B)
    outs = [out0]
    for k in range(3):
        t3k = jnp.sum(w3 * x1k[k][:, None, :], axis=0)
        outs.append(t2 * n[k] + t3k)
    outT_ref[...] = jnp.concatenate(outs, axis=0)


def _dense_edges(evT, xsT, w1s, w2ts, e_pad):
    nblk = e_pad // _BLOCK_E
    return pl.pallas_call(
        _dense_body,
        grid=(nblk,),
        in_specs=[
            pl.BlockSpec((3, _BLOCK_E), lambda i: (0, i)),
            pl.BlockSpec((128, _BLOCK_E), lambda i: (0, i)),
            pl.BlockSpec((_NUM_BASIS, 64), lambda i: (0, 0)),
            pl.BlockSpec((4096, 64), lambda i: (0, 0)),
        ],
        out_specs=pl.BlockSpec((128, _BLOCK_E), lambda i: (0, i)),
        out_shape=jax.ShapeDtypeStruct((128, e_pad), jnp.float32),
    )(evT, xsT, w1s, w2ts)


def _gather_rows(table, src3d, e_pad):
    per_w = e_pad // _NW
    nchunk = per_w // _SCAT_CHUNK
    mesh = plsc.VectorSubcoreMesh(core_axis_name="c", subcore_axis_name="s")

    nbuf = 4
    @functools.partial(
        pl.kernel, mesh=mesh,
        out_type=jax.ShapeDtypeStruct((e_pad, 128), jnp.float32),
        scratch_types=(
            [pltpu.VMEM((nchunk, _SCAT_CHUNK), jnp.int32)]
            + [pltpu.VMEM((_SCAT_CHUNK, 128), jnp.float32)] * nbuf
            + [pltpu.SemaphoreType.DMA] * (2 * nbuf)
        ),
    )
    def gat(table_hbm, src_hbm, out_hbm, idx_v, *rest):
        bufs = rest[:nbuf]
        gsem = rest[nbuf:2 * nbuf]
        wsem = rest[2 * nbuf:]
        c = jax.lax.axis_index("c")
        s = jax.lax.axis_index("s")
        wid = c * _NS + s
        pltpu.sync_copy(src_hbm.at[wid], idx_v)
        base = wid * per_w
        copies = [None] * nchunk
        writes = [None] * nchunk
        for j in range(min(nbuf, nchunk)):
            copies[j] = pltpu.async_copy(
                table_hbm.at[idx_v.at[j]], bufs[j % nbuf], gsem[j % nbuf])
        for j in range(nchunk):
            copies[j].wait()
            writes[j] = pltpu.async_copy(
                bufs[j % nbuf],
                out_hbm.at[pl.ds(base + j * _SCAT_CHUNK, _SCAT_CHUNK)],
                wsem[j % nbuf])
            k = j + nbuf
            if k < nchunk:
                writes[j].wait()
                copies[k] = pltpu.async_copy(
                    table_hbm.at[idx_v.at[k]], bufs[k % nbuf], gsem[k % nbuf])
        for j in range(max(0, nchunk - nbuf), nchunk):
            writes[j].wait()

    return gat(table, src3d)


def _scatter_add(summand, dst3d, zeros, e_pad):
    per_w = e_pad // _NW
    nchunk = per_w // _SCAT_CHUNK
    rpt = _N_PAD // _NS
    mesh = plsc.VectorSubcoreMesh(core_axis_name="c", subcore_axis_name="s")

    @functools.partial(
        pl.kernel, mesh=mesh,
        out_type=jax.ShapeDtypeStruct((_NC, _N_PAD, 128), jnp.float32),
        scratch_types=[
            pltpu.VMEM((nchunk, _SCAT_CHUNK), jnp.int32),
            pltpu.VMEM((_SCAT_CHUNK, 128), jnp.float32),
            pltpu.VMEM_SHARED((_N_PAD, 128), jnp.float32),
        ],
    )
    def scat(summand_hbm, dst_hbm, zeros_hbm, out_hbm, idx_v, data_v, acc):
        c = jax.lax.axis_index("c")
        s = jax.lax.axis_index("s")
        pltpu.sync_copy(zeros_hbm, acc.at[pl.ds(s * rpt, rpt)])
        plsc.subcore_barrier()
        wid = c * _NS + s
        pltpu.sync_copy(dst_hbm.at[wid], idx_v)
        base = wid * per_w
        for j in range(nchunk):
            pltpu.sync_copy(
                summand_hbm.at[pl.ds(base + j * _SCAT_CHUNK, _SCAT_CHUNK)], data_v)
            pltpu.sync_copy(data_v, acc.at[idx_v.at[j]], add=True)
        plsc.subcore_barrier()
        pltpu.sync_copy(acc.at[pl.ds(s * rpt, rpt)],
                        out_hbm.at[c, pl.ds(s * rpt, rpt)])

    return scat(summand, dst3d, zeros)


def _combine_body(a_ref, b_ref, o_ref):
    o_ref[...] = a_ref[...] + b_ref[...]


def _combine(parts):
    return pl.pallas_call(
        _combine_body,
        out_shape=jax.ShapeDtypeStruct((_N_PAD, 128), jnp.float32),
    )(parts[0], parts[1])


@jax.jit
def kernel(x, src, dst, edge_vec, W1, W2):
    N = x.shape[0]
    E = src.shape[0]
    grain = _NW * _SCAT_CHUNK
    e_pad = ((E + grain - 1) // grain) * grain

    c_out = (1.0 / (_M0 + _M1)) ** 0.5
    w1s = W1 * np.float32(1.14136 * np.exp(2.0))
    scale = np.float32((2.0 ** 0.5) / (64.0 ** 0.5) * c_out / (E / N))
    w2s = W2 * scale
    o1, o2 = _M0 * _M0, _M0 * _M0 + _M0 * _M1
    w2s = w2s.at[:, o1:o2].mul(np.float32(3.0 ** 0.5))
    w2ts = w2s.T.astype(jnp.bfloat16)

    u = np.arange(_M1)
    perm_in = np.concatenate(
        [np.arange(_M0)] + [_M0 + 3 * u + k for k in range(3)]).astype(np.int32)
    x_perm = x[:, perm_in]

    src3d = jnp.pad(src, (0, e_pad - E)).reshape(
        _NW, e_pad // (_NW * _SCAT_CHUNK), _SCAT_CHUNK)
    xs = _gather_rows(x_perm, src3d, e_pad)
    xsT = xs.T
    evT = jnp.pad(edge_vec[:, np.array([1, 2, 0])],
                  ((0, e_pad - E), (0, 0)), constant_values=1.0).T

    outT = _dense_edges(evT, xsT, w1s, w2ts, e_pad)

    summand = outT.T
    dst3d = jnp.pad(dst, (0, e_pad - E), constant_values=_N_PAD - 8
                    ).reshape(_NW, e_pad // (_NW * _SCAT_CHUNK), _SCAT_CHUNK)
    zeros = jnp.zeros((_N_PAD // _NS, 128), jnp.float32)
    parts = _scatter_add(summand, dst3d, zeros, e_pad)
    out_km = _combine(parts)[:N]

    perm_out = np.concatenate(
        [np.arange(_M0)] + [_M0 + 3 * u + k for k in range(3)]).astype(np.int32)
    inv = np.empty_like(perm_out)
    inv[perm_out] = np.arange(128, dtype=np.int32)
    return out_km[:, inv]


# speedup vs baseline: 6.3383x; 1.0933x over previous
import functools
import numpy as np
import jax
import jax.numpy as jnp
from jax.experimental import pallas as pl
from jax.experimental.pallas import tpu as pltpu
from jax.experimental.pallas import tpu_sc as plsc

_M0 = 32
_M1 = 32
_NUM_BASIS = 10
_MAX_RADIUS = 4.5
_BLOCK_E = 256

_NC = 2
_NS = 16
_NW = _NC * _NS
_SCAT_CHUNK = 128
_N_PAD = 10112


def _dense_body(evT_ref, xs_ref, w1_ref, w2t_ref, out_ref):
    evT = evT_ref[...]
    d2 = jnp.sum(evT * evT, axis=0, keepdims=True)
    d = jnp.sqrt(d2)
    step = _MAX_RADIUS / (_NUM_BASIS + 1)
    B = evT.shape[1]
    basis = (jax.lax.broadcasted_iota(jnp.int32, (_NUM_BASIS, B), 0) + 1
             ).astype(jnp.float32) * step
    diff = (d - basis) * (1.0 / step)

    def sus(t):
        ts = jnp.where(t > 0.0, t, 1.0)
        return jnp.where(t > 0.0, jnp.exp(-1.0 / ts), 0.0)

    g = sus(diff + 1.0) * sus(1.0 - diff)
    h = jax.nn.relu(
        jax.lax.dot_general(w1_ref[...], g, (((0,), (0,)), ((), ())),
                            preferred_element_type=jnp.float32))
    w = jax.lax.dot_general(w2t_ref[...], h.astype(jnp.bfloat16),
                            (((1,), (0,)), ((), ())),
                            preferred_element_type=jnp.float32)

    xsT = xs_ref[...].T
    x0 = xsT[:_M0]
    x1k = [xsT[_M0 + 32 * k:_M0 + 32 * (k + 1)] for k in range(3)]
    dinv = 1.0 / jnp.maximum(d, 1e-12)
    n = [evT[k:k + 1] * dinv for k in range(3)]
    dot = x1k[0] * n[0] + x1k[1] * n[1] + x1k[2] * n[2]

    def contract(wblk, vec):
        p = wblk.reshape(_M0, _M0, B) * vec[:, None, :]
        return jnp.sum(p, axis=0)

    t1 = contract(w[0:1024], x0)
    t2 = contract(w[1024:2048], x0)
    t4 = contract(w[3072:4096], dot)
    out0 = t1 + t4
    w3 = w[2048:3072].reshape(_M0, _M0, B)
    outs = [out0]
    for k in range(3):
        t3k = jnp.sum(w3 * x1k[k][:, None, :], axis=0)
        outs.append(t2 * n[k] + t3k)
    out_ref[...] = jnp.concatenate(outs, axis=0).T


def _dense_edges(evT, xs, w1s, w2ts, e_pad):
    nblk = e_pad // _BLOCK_E
    return pl.pallas_call(
        _dense_body,
        grid=(nblk,),
        in_specs=[
            pl.BlockSpec((3, _BLOCK_E), lambda i: (0, i)),
            pl.BlockSpec((_BLOCK_E, 128), lambda i: (i, 0)),
            pl.BlockSpec((_NUM_BASIS, 64), lambda i: (0, 0)),
            pl.BlockSpec((4096, 64), lambda i: (0, 0)),
        ],
        out_specs=pl.BlockSpec((_BLOCK_E, 128), lambda i: (i, 0)),
        out_shape=jax.ShapeDtypeStruct((e_pad, 128), jnp.float32),
    )(evT, xs, w1s, w2ts)


def _gather_rows(table, src3d, e_pad):
    per_w = e_pad // _NW
    nchunk = per_w // _SCAT_CHUNK
    mesh = plsc.VectorSubcoreMesh(core_axis_name="c", subcore_axis_name="s")

    nbuf = 4
    @functools.partial(
        pl.kernel, mesh=mesh,
        out_type=jax.ShapeDtypeStruct((e_pad, 128), jnp.float32),
        scratch_types=(
            [pltpu.VMEM((nchunk, _SCAT_CHUNK), jnp.int32)]
            + [pltpu.VMEM((_SCAT_CHUNK, 128), jnp.float32)] * nbuf
            + [pltpu.SemaphoreType.DMA] * (2 * nbuf)
        ),
    )
    def gat(table_hbm, src_hbm, out_hbm, idx_v, *rest):
        bufs = rest[:nbuf]
        gsem = rest[nbuf:2 * nbuf]
        wsem = rest[2 * nbuf:]
        c = jax.lax.axis_index("c")
        s = jax.lax.axis_index("s")
        wid = c * _NS + s
        pltpu.sync_copy(src_hbm.at[wid], idx_v)
        base = wid * per_w
        copies = [None] * nchunk
        writes = [None] * nchunk
        for j in range(min(nbuf, nchunk)):
            copies[j] = pltpu.async_copy(
                table_hbm.at[idx_v.at[j]], bufs[j % nbuf], gsem[j % nbuf])
        for j in range(nchunk):
            copies[j].wait()
            writes[j] = pltpu.async_copy(
                bufs[j % nbuf],
                out_hbm.at[pl.ds(base + j * _SCAT_CHUNK, _SCAT_CHUNK)],
                wsem[j % nbuf])
            k = j + nbuf
            if k < nchunk:
                writes[j].wait()
                copies[k] = pltpu.async_copy(
                    table_hbm.at[idx_v.at[k]], bufs[k % nbuf], gsem[k % nbuf])
        for j in range(max(0, nchunk - nbuf), nchunk):
            writes[j].wait()

    return gat(table, src3d)


def _scatter_add(summand, dst3d, zeros, e_pad):
    per_w = e_pad // _NW
    nchunk = per_w // _SCAT_CHUNK
    rpt = _N_PAD // _NS
    mesh = plsc.VectorSubcoreMesh(core_axis_name="c", subcore_axis_name="s")

    @functools.partial(
        pl.kernel, mesh=mesh,
        out_type=jax.ShapeDtypeStruct((_NC, _N_PAD, 128), jnp.float32),
        scratch_types=[
            pltpu.VMEM((nchunk, _SCAT_CHUNK), jnp.int32),
            pltpu.VMEM((_SCAT_CHUNK, 128), jnp.float32),
            pltpu.VMEM_SHARED((_N_PAD, 128), jnp.float32),
        ],
    )
    def scat(summand_hbm, dst_hbm, zeros_hbm, out_hbm, idx_v, data_v, acc):
        c = jax.lax.axis_index("c")
        s = jax.lax.axis_index("s")
        pltpu.sync_copy(zeros_hbm, acc.at[pl.ds(s * rpt, rpt)])
        plsc.subcore_barrier()
        wid = c * _NS + s
        pltpu.sync_copy(dst_hbm.at[wid], idx_v)
        base = wid * per_w
        for j in range(nchunk):
            pltpu.sync_copy(
                summand_hbm.at[pl.ds(base + j * _SCAT_CHUNK, _SCAT_CHUNK)], data_v)
            pltpu.sync_copy(data_v, acc.at[idx_v.at[j]], add=True)
        plsc.subcore_barrier()
        pltpu.sync_copy(acc.at[pl.ds(s * rpt, rpt)],
                        out_hbm.at[c, pl.ds(s * rpt, rpt)])

    return scat(summand, dst3d, zeros)


def _combine_body(a_ref, b_ref, o_ref):
    o_ref[...] = a_ref[...] + b_ref[...]


def _combine(parts):
    return pl.pallas_call(
        _combine_body,
        out_shape=jax.ShapeDtypeStruct((_N_PAD, 128), jnp.float32),
    )(parts[0], parts[1])


@jax.jit
def kernel(x, src, dst, edge_vec, W1, W2):
    N = x.shape[0]
    E = src.shape[0]
    grain = _NW * _SCAT_CHUNK
    e_pad = ((E + grain - 1) // grain) * grain

    c_out = (1.0 / (_M0 + _M1)) ** 0.5
    w1s = W1 * np.float32(1.14136 * np.exp(2.0))
    scale = np.float32((2.0 ** 0.5) / (64.0 ** 0.5) * c_out / (E / N))
    w2s = W2 * scale
    o1, o2 = _M0 * _M0, _M0 * _M0 + _M0 * _M1
    w2s = w2s.at[:, o1:o2].mul(np.float32(3.0 ** 0.5))
    w2ts = w2s.T.astype(jnp.bfloat16)

    u = np.arange(_M1)
    perm_in = np.concatenate(
        [np.arange(_M0)] + [_M0 + 3 * u + k for k in range(3)]).astype(np.int32)
    x_perm = x[:, perm_in]

    src3d = jnp.pad(src, (0, e_pad - E)).reshape(
        _NW, e_pad // (_NW * _SCAT_CHUNK), _SCAT_CHUNK)
    xs = _gather_rows(x_perm, src3d, e_pad)
    evT = jnp.pad(edge_vec[:, np.array([1, 2, 0])],
                  ((0, e_pad - E), (0, 0)), constant_values=1.0).T

    summand = _dense_edges(evT, xs, w1s, w2ts, e_pad)

    dst3d = jnp.pad(dst, (0, e_pad - E), constant_values=_N_PAD - 8
                    ).reshape(_NW, e_pad // (_NW * _SCAT_CHUNK), _SCAT_CHUNK)
    zeros = jnp.zeros((_N_PAD // _NS, 128), jnp.float32)
    parts = _scatter_add(summand, dst3d, zeros, e_pad)
    out_km = _combine(parts)[:N]

    perm_out = np.concatenate(
        [np.arange(_M0)] + [_M0 + 3 * u + k for k in range(3)]).astype(np.int32)
    inv = np.empty_like(perm_out)
    inv[perm_out] = np.arange(128, dtype=np.int32)
    return out_km[:, inv]


# speedup vs baseline: 6.7286x; 1.0616x over previous
import functools
import numpy as np
import jax
import jax.numpy as jnp
from jax.experimental import pallas as pl
from jax.experimental.pallas import tpu as pltpu
from jax.experimental.pallas import tpu_sc as plsc

_M0 = 32
_M1 = 32
_NUM_BASIS = 10
_MAX_RADIUS = 4.5
_BLOCK_E = 512

_NC = 2
_NS = 16
_NW = _NC * _NS
_SCAT_CHUNK = 128
_N_PAD = 10112


def _dense_body(evT_ref, xs_ref, w1_ref, w2t_ref, out_ref):
    evT = evT_ref[...]
    d2 = jnp.sum(evT * evT, axis=0, keepdims=True)
    d = jnp.sqrt(d2)
    step = _MAX_RADIUS / (_NUM_BASIS + 1)
    B = evT.shape[1]
    basis = (jax.lax.broadcasted_iota(jnp.int32, (_NUM_BASIS, B), 0) + 1
             ).astype(jnp.float32) * step
    diff = (d - basis) * (1.0 / step)

    def sus(t):
        ts = jnp.where(t > 0.0, t, 1.0)
        return jnp.where(t > 0.0, jnp.exp(-1.0 / ts), 0.0)

    g = sus(diff + 1.0) * sus(1.0 - diff)
    h = jax.nn.relu(
        jax.lax.dot_general(w1_ref[...], g, (((0,), (0,)), ((), ())),
                            preferred_element_type=jnp.float32))
    w = jax.lax.dot_general(w2t_ref[...], h.astype(jnp.bfloat16),
                            (((1,), (0,)), ((), ())),
                            preferred_element_type=jnp.float32)

    xsT = xs_ref[...].T
    x0 = xsT[:_M0]
    x1k = [xsT[_M0 + 32 * k:_M0 + 32 * (k + 1)] for k in range(3)]
    dinv = 1.0 / jnp.maximum(d, 1e-12)
    n = [evT[k:k + 1] * dinv for k in range(3)]
    dot = x1k[0] * n[0] + x1k[1] * n[1] + x1k[2] * n[2]

    def contract(wblk, vec):
        p = wblk.reshape(_M0, _M0, B) * vec[:, None, :]
        return jnp.sum(p, axis=0)

    t1 = contract(w[0:1024], x0)
    t2 = contract(w[1024:2048], x0)
    t4 = contract(w[3072:4096], dot)
    out0 = t1 + t4
    w3 = w[2048:3072].reshape(_M0, _M0, B)
    outs = [out0]
    for k in range(3):
        t3k = jnp.sum(w3 * x1k[k][:, None, :], axis=0)
        outs.append(t2 * n[k] + t3k)
    out_ref[...] = jnp.concatenate(outs, axis=0).T


def _dense_edges(evT, xs, w1s, w2ts, e_pad):
    nblk = e_pad // _BLOCK_E
    return pl.pallas_call(
        _dense_body,
        grid=(nblk,),
        in_specs=[
            pl.BlockSpec((3, _BLOCK_E), lambda i: (0, i)),
            pl.BlockSpec((_BLOCK_E, 128), lambda i: (i, 0)),
            pl.BlockSpec((_NUM_BASIS, 64), lambda i: (0, 0)),
            pl.BlockSpec((4096, 64), lambda i: (0, 0)),
        ],
        out_specs=pl.BlockSpec((_BLOCK_E, 128), lambda i: (i, 0)),
        out_shape=jax.ShapeDtypeStruct((e_pad, 128), jnp.float32),
    )(evT, xs, w1s, w2ts)


def _gather_rows(table, src3d, e_pad):
    per_w = e_pad // _NW
    nchunk = per_w // _SCAT_CHUNK
    mesh = plsc.VectorSubcoreMesh(core_axis_name="c", subcore_axis_name="s")

    nbuf = 4
    @functools.partial(
        pl.kernel, mesh=mesh,
        out_type=jax.ShapeDtypeStruct((e_pad, 128), jnp.float32),
        scratch_types=(
            [pltpu.VMEM((nchunk, _SCAT_CHUNK), jnp.int32)]
            + [pltpu.VMEM((_SCAT_CHUNK, 128), jnp.float32)] * nbuf
            + [pltpu.SemaphoreType.DMA] * (2 * nbuf)
        ),
    )
    def gat(table_hbm, src_hbm, out_hbm, idx_v, *rest):
        bufs = rest[:nbuf]
        gsem = rest[nbuf:2 * nbuf]
        wsem = rest[2 * nbuf:]
        c = jax.lax.axis_index("c")
        s = jax.lax.axis_index("s")
        wid = c * _NS + s
        pltpu.sync_copy(src_hbm.at[wid], idx_v)
        base = wid * per_w
        copies = [None] * nchunk
        writes = [None] * nchunk
        for j in range(min(nbuf, nchunk)):
            copies[j] = pltpu.async_copy(
                table_hbm.at[idx_v.at[j]], bufs[j % nbuf], gsem[j % nbuf])
        for j in range(nchunk):
            copies[j].wait()
            writes[j] = pltpu.async_copy(
                bufs[j % nbuf],
                out_hbm.at[pl.ds(base + j * _SCAT_CHUNK, _SCAT_CHUNK)],
                wsem[j % nbuf])
            k = j + nbuf
            if k < nchunk:
                writes[j].wait()
                copies[k] = pltpu.async_copy(
                    table_hbm.at[idx_v.at[k]], bufs[k % nbuf], gsem[k % nbuf])
        for j in range(max(0, nchunk - nbuf), nchunk):
            writes[j].wait()

    return gat(table, src3d)


def _scatter_add(summand, dst3d, zeros, e_pad):
    per_w = e_pad // _NW
    nchunk = per_w // _SCAT_CHUNK
    rpt = _N_PAD // _NS
    mesh = plsc.VectorSubcoreMesh(core_axis_name="c", subcore_axis_name="s")

    @functools.partial(
        pl.kernel, mesh=mesh,
        out_type=jax.ShapeDtypeStruct((_NC, _N_PAD, 128), jnp.float32),
        scratch_types=[
            pltpu.VMEM((nchunk, _SCAT_CHUNK), jnp.int32),
            pltpu.VMEM((_SCAT_CHUNK, 128), jnp.float32),
            pltpu.VMEM_SHARED((_N_PAD, 128), jnp.float32),
        ],
    )
    def scat(summand_hbm, dst_hbm, zeros_hbm, out_hbm, idx_v, data_v, acc):
        c = jax.lax.axis_index("c")
        s = jax.lax.axis_index("s")
        pltpu.sync_copy(zeros_hbm, acc.at[pl.ds(s * rpt, rpt)])
        plsc.subcore_barrier()
        wid = c * _NS + s
        pltpu.sync_copy(dst_hbm.at[wid], idx_v)
        base = wid * per_w
        for j in range(nchunk):
            pltpu.sync_copy(
                summand_hbm.at[pl.ds(base + j * _SCAT_CHUNK, _SCAT_CHUNK)], data_v)
            pltpu.sync_copy(data_v, acc.at[idx_v.at[j]], add=True)
        plsc.subcore_barrier()
        pltpu.sync_copy(acc.at[pl.ds(s * rpt, rpt)],
                        out_hbm.at[c, pl.ds(s * rpt, rpt)])

    return scat(summand, dst3d, zeros)


def _combine_body(a_ref, b_ref, o_ref):
    o_ref[...] = a_ref[...] + b_ref[...]


def _combine(parts):
    return pl.pallas_call(
        _combine_body,
        out_shape=jax.ShapeDtypeStruct((_N_PAD, 128), jnp.float32),
    )(parts[0], parts[1])


@jax.jit
def kernel(x, src, dst, edge_vec, W1, W2):
    N = x.shape[0]
    E = src.shape[0]
    grain = _NW * _SCAT_CHUNK
    e_pad = ((E + grain - 1) // grain) * grain

    c_out = (1.0 / (_M0 + _M1)) ** 0.5
    w1s = W1 * np.float32(1.14136 * np.exp(2.0))
    scale = np.float32((2.0 ** 0.5) / (64.0 ** 0.5) * c_out / (E / N))
    w2s = W2 * scale
    o1, o2 = _M0 * _M0, _M0 * _M0 + _M0 * _M1
    w2s = w2s.at[:, o1:o2].mul(np.float32(3.0 ** 0.5))
    w2ts = w2s.T.astype(jnp.bfloat16)

    u = np.arange(_M1)
    perm_in = np.concatenate(
        [np.arange(_M0)] + [_M0 + 3 * u + k for k in range(3)]).astype(np.int32)
    x_perm = x[:, perm_in]

    src3d = jnp.pad(src, (0, e_pad - E)).reshape(
        _NW, e_pad // (_NW * _SCAT_CHUNK), _SCAT_CHUNK)
    xs = _gather_rows(x_perm, src3d, e_pad)
    evT = jnp.pad(edge_vec[:, np.array([1, 2, 0])],
                  ((0, e_pad - E), (0, 0)), constant_values=1.0).T

    summand = _dense_edges(evT, xs, w1s, w2ts, e_pad)

    dst3d = jnp.pad(dst, (0, e_pad - E), constant_values=_N_PAD - 8
                    ).reshape(_NW, e_pad // (_NW * _SCAT_CHUNK), _SCAT_CHUNK)
    zeros = jnp.zeros((_N_PAD // _NS, 128), jnp.float32)
    parts = _scatter_add(summand, dst3d, zeros, e_pad)
    out_km = _combine(parts)[:N]

    perm_out = np.concatenate(
        [np.arange(_M0)] + [_M0 + 3 * u + k for k in range(3)]).astype(np.int32)
    inv = np.empty_like(perm_out)
    inv[perm_out] = np.arange(128, dtype=np.int32)
    return out_km[:, inv]


# speedup vs baseline: 7.1988x; 1.0699x over previous
import functools
import numpy as np
import jax
import jax.numpy as jnp
from jax.experimental import pallas as pl
from jax.experimental.pallas import tpu as pltpu
from jax.experimental.pallas import tpu_sc as plsc

_M0 = 32
_M1 = 32
_NUM_BASIS = 10
_MAX_RADIUS = 4.5
_BLOCK_E = 1024

_NC = 2
_NS = 16
_NW = _NC * _NS
_SCAT_CHUNK = 128
_N_PAD = 10112


def _dense_body(evT_ref, xs_ref, w1_ref, w2t_ref, out_ref):
    evT = evT_ref[...]
    d2 = jnp.sum(evT * evT, axis=0, keepdims=True)
    d = jnp.sqrt(d2)
    step = _MAX_RADIUS / (_NUM_BASIS + 1)
    B = evT.shape[1]
    basis = (jax.lax.broadcasted_iota(jnp.int32, (_NUM_BASIS, B), 0) + 1
             ).astype(jnp.float32) * step
    diff = (d - basis) * (1.0 / step)

    def sus(t):
        ts = jnp.where(t > 0.0, t, 1.0)
        return jnp.where(t > 0.0, jnp.exp(-1.0 / ts), 0.0)

    g = sus(diff + 1.0) * sus(1.0 - diff)
    h = jax.nn.relu(
        jax.lax.dot_general(w1_ref[...], g, (((0,), (0,)), ((), ())),
                            preferred_element_type=jnp.float32))
    w = jax.lax.dot_general(w2t_ref[...], h.astype(jnp.bfloat16),
                            (((1,), (0,)), ((), ())),
                            preferred_element_type=jnp.float32)

    xsT = xs_ref[...].T
    x0 = xsT[:_M0]
    x1k = [xsT[_M0 + 32 * k:_M0 + 32 * (k + 1)] for k in range(3)]
    dinv = 1.0 / jnp.maximum(d, 1e-12)
    n = [evT[k:k + 1] * dinv for k in range(3)]
    dot = x1k[0] * n[0] + x1k[1] * n[1] + x1k[2] * n[2]

    def contract(wblk, vec):
        p = wblk.reshape(_M0, _M0, B) * vec[:, None, :]
        return jnp.sum(p, axis=0)

    t1 = contract(w[0:1024], x0)
    t2 = contract(w[1024:2048], x0)
    t4 = contract(w[3072:4096], dot)
    out0 = t1 + t4
    w3 = w[2048:3072].reshape(_M0, _M0, B)
    outs = [out0]
    for k in range(3):
        t3k = jnp.sum(w3 * x1k[k][:, None, :], axis=0)
        outs.append(t2 * n[k] + t3k)
    out_ref[...] = jnp.concatenate(outs, axis=0).T


def _dense_edges(evT, xs, w1s, w2ts, e_pad):
    nblk = e_pad // _BLOCK_E
    return pl.pallas_call(
        _dense_body,
        grid=(nblk,),
        in_specs=[
            pl.BlockSpec((3, _BLOCK_E), lambda i: (0, i)),
            pl.BlockSpec((_BLOCK_E, 128), lambda i: (i, 0)),
            pl.BlockSpec((_NUM_BASIS, 64), lambda i: (0, 0)),
            pl.BlockSpec((4096, 64), lambda i: (0, 0)),
        ],
        out_specs=pl.BlockSpec((_BLOCK_E, 128), lambda i: (i, 0)),
        out_shape=jax.ShapeDtypeStruct((e_pad, 128), jnp.float32),
    )(evT, xs, w1s, w2ts)


def _gather_rows(table, src3d, e_pad):
    per_w = e_pad // _NW
    nchunk = per_w // _SCAT_CHUNK
    mesh = plsc.VectorSubcoreMesh(core_axis_name="c", subcore_axis_name="s")

    nbuf = 4
    @functools.partial(
        pl.kernel, mesh=mesh,
        out_type=jax.ShapeDtypeStruct((e_pad, 128), jnp.float32),
        scratch_types=(
            [pltpu.VMEM((nchunk, _SCAT_CHUNK), jnp.int32)]
            + [pltpu.VMEM((_SCAT_CHUNK, 128), jnp.float32)] * nbuf
            + [pltpu.SemaphoreType.DMA] * (2 * nbuf)
        ),
    )
    def gat(table_hbm, src_hbm, out_hbm, idx_v, *rest):
        bufs = rest[:nbuf]
        gsem = rest[nbuf:2 * nbuf]
        wsem = rest[2 * nbuf:]
        c = jax.lax.axis_index("c")
        s = jax.lax.axis_index("s")
        wid = c * _NS + s
        pltpu.sync_copy(src_hbm.at[wid], idx_v)
        base = wid * per_w
        copies = [None] * nchunk
        writes = [None] * nchunk
        for j in range(min(nbuf, nchunk)):
            copies[j] = pltpu.async_copy(
                table_hbm.at[idx_v.at[j]], bufs[j % nbuf], gsem[j % nbuf])
        for j in range(nchunk):
            copies[j].wait()
            writes[j] = pltpu.async_copy(
                bufs[j % nbuf],
                out_hbm.at[pl.ds(base + j * _SCAT_CHUNK, _SCAT_CHUNK)],
                wsem[j % nbuf])
            k = j + nbuf
            if k < nchunk:
                writes[j].wait()
                copies[k] = pltpu.async_copy(
                    table_hbm.at[idx_v.at[k]], bufs[k % nbuf], gsem[k % nbuf])
        for j in range(max(0, nchunk - nbuf), nchunk):
            writes[j].wait()

    return gat(table, src3d)


def _scatter_add(summand, dst3d, zeros, e_pad):
    per_w = e_pad // _NW
    nchunk = per_w // _SCAT_CHUNK
    rpt = _N_PAD // _NS
    mesh = plsc.VectorSubcoreMesh(core_axis_name="c", subcore_axis_name="s")

    @functools.partial(
        pl.kernel, mesh=mesh,
        out_type=jax.ShapeDtypeStruct((_NC, _N_PAD, 128), jnp.float32),
        scratch_types=[
            pltpu.VMEM((nchunk, _SCAT_CHUNK), jnp.int32),
            pltpu.VMEM((_SCAT_CHUNK, 128), jnp.float32),
            pltpu.VMEM_SHARED((_N_PAD, 128), jnp.float32),
        ],
    )
    def scat(summand_hbm, dst_hbm, zeros_hbm, out_hbm, idx_v, data_v, acc):
        c = jax.lax.axis_index("c")
        s = jax.lax.axis_index("s")
        pltpu.sync_copy(zeros_hbm, acc.at[pl.ds(s * rpt, rpt)])
        plsc.subcore_barrier()
        wid = c * _NS + s
        pltpu.sync_copy(dst_hbm.at[wid], idx_v)
        base = wid * per_w
        for j in range(nchunk):
            pltpu.sync_copy(
                summand_hbm.at[pl.ds(base + j * _SCAT_CHUNK, _SCAT_CHUNK)], data_v)
            pltpu.sync_copy(data_v, acc.at[idx_v.at[j]], add=True)
        plsc.subcore_barrier()
        pltpu.sync_copy(acc.at[pl.ds(s * rpt, rpt)],
                        out_hbm.at[c, pl.ds(s * rpt, rpt)])

    return scat(summand, dst3d, zeros)


def _combine_body(a_ref, b_ref, o_ref):
    o_ref[...] = a_ref[...] + b_ref[...]


def _combine(parts):
    return pl.pallas_call(
        _combine_body,
        out_shape=jax.ShapeDtypeStruct((_N_PAD, 128), jnp.float32),
    )(parts[0], parts[1])


@jax.jit
def kernel(x, src, dst, edge_vec, W1, W2):
    N = x.shape[0]
    E = src.shape[0]
    grain = _NW * _SCAT_CHUNK
    e_pad = ((E + grain - 1) // grain) * grain

    c_out = (1.0 / (_M0 + _M1)) ** 0.5
    w1s = W1 * np.float32(1.14136 * np.exp(2.0))
    scale = np.float32((2.0 ** 0.5) / (64.0 ** 0.5) * c_out / (E / N))
    w2s = W2 * scale
    o1, o2 = _M0 * _M0, _M0 * _M0 + _M0 * _M1
    w2s = w2s.at[:, o1:o2].mul(np.float32(3.0 ** 0.5))
    w2ts = w2s.T.astype(jnp.bfloat16)

    u = np.arange(_M1)
    perm_in = np.concatenate(
        [np.arange(_M0)] + [_M0 + 3 * u + k for k in range(3)]).astype(np.int32)
    x_perm = x[:, perm_in]

    src3d = jnp.pad(src, (0, e_pad - E)).reshape(
        _NW, e_pad // (_NW * _SCAT_CHUNK), _SCAT_CHUNK)
    xs = _gather_rows(x_perm, src3d, e_pad)
    evT = jnp.pad(edge_vec[:, np.array([1, 2, 0])],
                  ((0, e_pad - E), (0, 0)), constant_values=1.0).T

    summand = _dense_edges(evT, xs, w1s, w2ts, e_pad)

    dst3d = jnp.pad(dst, (0, e_pad - E), constant_values=_N_PAD - 8
                    ).reshape(_NW, e_pad // (_NW * _SCAT_CHUNK), _SCAT_CHUNK)
    zeros = jnp.zeros((_N_PAD // _NS, 128), jnp.float32)
    parts = _scatter_add(summand, dst3d, zeros, e_pad)
    out_km = _combine(parts)[:N]

    perm_out = np.concatenate(
        [np.arange(_M0)] + [_M0 + 3 * u + k for k in range(3)]).astype(np.int32)
    inv = np.empty_like(perm_out)
    inv[perm_out] = np.arange(128, dtype=np.int32)
    return out_km[:, inv]


# speedup vs baseline: 7.5871x; 1.0539x over previous
import functools
import numpy as np
import jax
import jax.numpy as jnp
from jax.experimental import pallas as pl
from jax.experimental.pallas import tpu as pltpu
from jax.experimental.pallas import tpu_sc as plsc

_M0 = 32
_M1 = 32
_NUM_BASIS = 10
_MAX_RADIUS = 4.5
_BLOCK_E = 2048

_NC = 2
_NS = 16
_NW = _NC * _NS
_SCAT_CHUNK = 128
_N_PAD = 10112


def _dense_body(evT_ref, xs_ref, w1_ref, w2t_ref, out_ref):
    evT = evT_ref[...]
    d2 = jnp.sum(evT * evT, axis=0, keepdims=True)
    d = jnp.sqrt(d2)
    step = _MAX_RADIUS / (_NUM_BASIS + 1)
    B = evT.shape[1]
    basis = (jax.lax.broadcasted_iota(jnp.int32, (_NUM_BASIS, B), 0) + 1
             ).astype(jnp.float32) * step
    diff = (d - basis) * (1.0 / step)

    def sus(t):
        ts = jnp.where(t > 0.0, t, 1.0)
        return jnp.where(t > 0.0, jnp.exp(-1.0 / ts), 0.0)

    g = sus(diff + 1.0) * sus(1.0 - diff)
    h = jax.nn.relu(
        jax.lax.dot_general(w1_ref[...], g, (((0,), (0,)), ((), ())),
                            preferred_element_type=jnp.float32))
    w = jax.lax.dot_general(w2t_ref[...], h.astype(jnp.bfloat16),
                            (((1,), (0,)), ((), ())),
                            preferred_element_type=jnp.float32)

    xsT = xs_ref[...].T
    x0 = xsT[:_M0]
    x1k = [xsT[_M0 + 32 * k:_M0 + 32 * (k + 1)] for k in range(3)]
    dinv = 1.0 / jnp.maximum(d, 1e-12)
    n = [evT[k:k + 1] * dinv for k in range(3)]
    dot = x1k[0] * n[0] + x1k[1] * n[1] + x1k[2] * n[2]

    def contract(wblk, vec):
        p = wblk.reshape(_M0, _M0, B) * vec[:, None, :]
        return jnp.sum(p, axis=0)

    t1 = contract(w[0:1024], x0)
    t2 = contract(w[1024:2048], x0)
    t4 = contract(w[3072:4096], dot)
    out0 = t1 + t4
    w3 = w[2048:3072].reshape(_M0, _M0, B)
    outs = [out0]
    for k in range(3):
        t3k = jnp.sum(w3 * x1k[k][:, None, :], axis=0)
        outs.append(t2 * n[k] + t3k)
    out_ref[...] = jnp.concatenate(outs, axis=0).T


def _dense_edges(evT, xs, w1s, w2ts, e_pad):
    nblk = e_pad // _BLOCK_E
    return pl.pallas_call(
        _dense_body,
        grid=(nblk,),
        in_specs=[
            pl.BlockSpec((3, _BLOCK_E), lambda i: (0, i)),
            pl.BlockSpec((_BLOCK_E, 128), lambda i: (i, 0)),
            pl.BlockSpec((_NUM_BASIS, 64), lambda i: (0, 0)),
            pl.BlockSpec((4096, 64), lambda i: (0, 0)),
        ],
        out_specs=pl.BlockSpec((_BLOCK_E, 128), lambda i: (i, 0)),
        out_shape=jax.ShapeDtypeStruct((e_pad, 128), jnp.float32),
    )(evT, xs, w1s, w2ts)


def _gather_rows(table, src3d, e_pad):
    per_w = e_pad // _NW
    nchunk = per_w // _SCAT_CHUNK
    mesh = plsc.VectorSubcoreMesh(core_axis_name="c", subcore_axis_name="s")

    nbuf = 4
    @functools.partial(
        pl.kernel, mesh=mesh,
        out_type=jax.ShapeDtypeStruct((e_pad, 128), jnp.float32),
        scratch_types=(
            [pltpu.VMEM((nchunk, _SCAT_CHUNK), jnp.int32)]
            + [pltpu.VMEM((_SCAT_CHUNK, 128), jnp.float32)] * nbuf
            + [pltpu.SemaphoreType.DMA] * (2 * nbuf)
        ),
    )
    def gat(table_hbm, src_hbm, out_hbm, idx_v, *rest):
        bufs = rest[:nbuf]
        gsem = rest[nbuf:2 * nbuf]
        wsem = rest[2 * nbuf:]
        c = jax.lax.axis_index("c")
        s = jax.lax.axis_index("s")
        wid = c * _NS + s
        pltpu.sync_copy(src_hbm.at[wid], idx_v)
        base = wid * per_w
        copies = [None] * nchunk
        writes = [None] * nchunk
        for j in range(min(nbuf, nchunk)):
            copies[j] = pltpu.async_copy(
                table_hbm.at[idx_v.at[j]], bufs[j % nbuf], gsem[j % nbuf])
        for j in range(nchunk):
            copies[j].wait()
            writes[j] = pltpu.async_copy(
                bufs[j % nbuf],
                out_hbm.at[pl.ds(base + j * _SCAT_CHUNK, _SCAT_CHUNK)],
                wsem[j % nbuf])
            k = j + nbuf
            if k < nchunk:
                writes[j].wait()
                copies[k] = pltpu.async_copy(
                    table_hbm.at[idx_v.at[k]], bufs[k % nbuf], gsem[k % nbuf])
        for j in range(max(0, nchunk - nbuf), nchunk):
            writes[j].wait()

    return gat(table, src3d)


def _scatter_add(summand, dst3d, zeros, e_pad):
    per_w = e_pad // _NW
    nchunk = per_w // _SCAT_CHUNK
    rpt = _N_PAD // _NS
    mesh = plsc.VectorSubcoreMesh(core_axis_name="c", subcore_axis_name="s")

    @functools.partial(
        pl.kernel, mesh=mesh,
        out_type=jax.ShapeDtypeStruct((_NC, _N_PAD, 128), jnp.float32),
        scratch_types=[
            pltpu.VMEM((nchunk, _SCAT_CHUNK), jnp.int32),
            pltpu.VMEM((_SCAT_CHUNK, 128), jnp.float32),
            pltpu.VMEM_SHARED((_N_PAD, 128), jnp.float32),
        ],
    )
    def scat(summand_hbm, dst_hbm, zeros_hbm, out_hbm, idx_v, data_v, acc):
        c = jax.lax.axis_index("c")
        s = jax.lax.axis_index("s")
        pltpu.sync_copy(zeros_hbm, acc.at[pl.ds(s * rpt, rpt)])
        plsc.subcore_barrier()
        wid = c * _NS + s
        pltpu.sync_copy(dst_hbm.at[wid], idx_v)
        base = wid * per_w
        for j in range(nchunk):
            pltpu.sync_copy(
                summand_hbm.at[pl.ds(base + j * _SCAT_CHUNK, _SCAT_CHUNK)], data_v)
            pltpu.sync_copy(data_v, acc.at[idx_v.at[j]], add=True)
        plsc.subcore_barrier()
        pltpu.sync_copy(acc.at[pl.ds(s * rpt, rpt)],
                        out_hbm.at[c, pl.ds(s * rpt, rpt)])

    return scat(summand, dst3d, zeros)


def _combine_body(a_ref, b_ref, o_ref):
    o_ref[...] = a_ref[...] + b_ref[...]


def _combine(parts):
    return pl.pallas_call(
        _combine_body,
        out_shape=jax.ShapeDtypeStruct((_N_PAD, 128), jnp.float32),
    )(parts[0], parts[1])


@jax.jit
def kernel(x, src, dst, edge_vec, W1, W2):
    N = x.shape[0]
    E = src.shape[0]
    grain = _NW * _SCAT_CHUNK
    e_pad = ((E + grain - 1) // grain) * grain

    c_out = (1.0 / (_M0 + _M1)) ** 0.5
    w1s = W1 * np.float32(1.14136 * np.exp(2.0))
    scale = np.float32((2.0 ** 0.5) / (64.0 ** 0.5) * c_out / (E / N))
    w2s = W2 * scale
    o1, o2 = _M0 * _M0, _M0 * _M0 + _M0 * _M1
    w2s = w2s.at[:, o1:o2].mul(np.float32(3.0 ** 0.5))
    w2ts = w2s.T.astype(jnp.bfloat16)

    u = np.arange(_M1)
    perm_in = np.concatenate(
        [np.arange(_M0)] + [_M0 + 3 * u + k for k in range(3)]).astype(np.int32)
    x_perm = x[:, perm_in]

    src3d = jnp.pad(src, (0, e_pad - E)).reshape(
        _NW, e_pad // (_NW * _SCAT_CHUNK), _SCAT_CHUNK)
    xs = _gather_rows(x_perm, src3d, e_pad)
    evT = jnp.pad(edge_vec[:, np.array([1, 2, 0])],
                  ((0, e_pad - E), (0, 0)), constant_values=1.0).T

    summand = _dense_edges(evT, xs, w1s, w2ts, e_pad)

    dst3d = jnp.pad(dst, (0, e_pad - E), constant_values=_N_PAD - 8
                    ).reshape(_NW, e_pad // (_NW * _SCAT_CHUNK), _SCAT_CHUNK)
    zeros = jnp.zeros((_N_PAD // _NS, 128), jnp.float32)
    parts = _scatter_add(summand, dst3d, zeros, e_pad)
    out_km = _combine(parts)[:N]

    perm_out = np.concatenate(
        [np.arange(_M0)] + [_M0 + 3 * u + k for k in range(3)]).astype(np.int32)
    inv = np.empty_like(perm_out)
    inv[perm_out] = np.arange(128, dtype=np.int32)
    return out_km[:, inv]
